# Initial kernel scaffold; baseline (speedup 1.0000x reference)
import jax
import jax.numpy as jnp
from jax.experimental import pallas as pl


def kernel(x, edge_index, init_w1, w1, root_w1, b1, init_w2, w2, root_w2, b2):
    raise NotImplementedError("write your pallas kernel here")


# speedup vs baseline: 31.0647x; 31.0647x over previous
import functools

import jax
import jax.numpy as jnp
from jax import lax
from jax.experimental import pallas as pl
from jax.experimental.pallas import tpu as pltpu
from jax.experimental.pallas import tpu_sc as plsc

N = 10000
E = 320000
F_IN = 128
F_HID = 128
F_OUT = 64
K = 3

NC = 2
NS = 16
B = 128
EP = 323584
N_ACC = 10240
DEG_W = 16

@functools.cache
def _mesh():
    return plsc.VectorSubcoreMesh(core_axis_name="c", subcore_axis_name="s",
                                  num_cores=NC, num_subcores=NS)


@functools.cache
def _make_deg():
    return pl.kernel(
        _deg_body,
        out_type=(jax.ShapeDtypeStruct((N_ACC, DEG_W), jnp.float32),
                  jax.ShapeDtypeStruct((N_ACC, DEG_W), jnp.float32)),
        mesh=_mesh(),
        scratch_types=[
            pltpu.VMEM_SHARED((N_ACC, DEG_W), jnp.float32),
            pltpu.VMEM((B,), jnp.int32),
            pltpu.VMEM((B, DEG_W), jnp.float32),
            pltpu.SemaphoreType.DMA,
        ],
        compiler_params=pltpu.CompilerParams(use_tc_tiling_on_sc=False),
    )


def _deg(col_p):
    return _make_deg()(col_p)


def _deg_body(col_hbm, deg0, deg1, acc, colbuf, vals, sem):
    c = lax.axis_index("c")
    s = lax.axis_index("s")
    npt = EP // (NC * NS)
    nb = npt // B
    rpt = N_ACC // NS

    wpr = DEG_W // 16

    def fill(v):
        def body(i, _):
            vals[i // wpr, pl.ds((i % wpr) * 16, 16)] = jnp.full((16,), v, jnp.float32)
            return 0
        lax.fori_loop(0, B * wpr, body, 0)

    fill(0.0)
    r0 = s * rpt
    for j in range(rpt // B):
        pltpu.sync_copy(vals, acc.at[pl.ds(r0 + j * B, B)])
    fill(1.0)
    plsc.subcore_barrier()

    for cc in range(NC):
        @pl.when(c == cc)
        def _():
            base = (cc * NS + s) * npt

            def step(j, _):
                off = base + j * B
                pltpu.sync_copy(col_hbm.at[pl.ds(off, B)], colbuf)
                pltpu.sync_copy(vals, acc.at[colbuf], add=True)
                return 0
            lax.fori_loop(0, nb, step, 0)
    plsc.subcore_barrier()

    for cc in range(NC):
        @pl.when(c == cc)
        def _():
            out = (deg0, deg1)[cc]
            pltpu.sync_copy(acc.at[pl.ds(r0, rpt)], out.at[pl.ds(r0, rpt)])


@functools.cache
def _make_spmm(Wc):
    npt = EP // NS
    nb = npt // B
    rpt = N_ACC // NS

    def spmm(src0, src1, row_hbm, col_hbm, out0, out1,
             acc, rowbuf, colbuf, rows, sem):
        c = lax.axis_index("c")
        s = lax.axis_index("s")
        wpr = Wc // 16

        def zstep(i, _):
            rows[i // wpr, pl.ds((i % wpr) * 16, 16)] = jnp.zeros((16,), jnp.float32)
            return 0
        lax.fori_loop(0, B * wpr, zstep, 0)
        r0 = s * rpt
        for j in range(rpt // B):
            pltpu.sync_copy(rows, acc.at[pl.ds(r0 + j * B, B)])
        plsc.subcore_barrier()

        base = s * npt
        for cc in range(NC):
            @pl.when(c == cc)
            def _():
                src = (src0, src1)[cc]

                def step(j, _):
                    off = base + j * B
                    pltpu.sync_copy(row_hbm.at[pl.ds(off, B)], rowbuf)
                    pltpu.sync_copy(col_hbm.at[pl.ds(off, B)], colbuf)
                    pltpu.async_copy(src.at[rowbuf], rows, sem).wait()
                    pltpu.sync_copy(rows, acc.at[colbuf], add=True)
                    return 0
                lax.fori_loop(0, nb, step, 0)
        plsc.subcore_barrier()

        for cc in range(NC):
            @pl.when(c == cc)
            def _():
                out = (out0, out1)[cc]
                pltpu.sync_copy(acc.at[pl.ds(r0, rpt)], out.at[pl.ds(r0, rpt)])

    return pl.kernel(
        spmm,
        out_type=(jax.ShapeDtypeStruct((N_ACC, Wc), jnp.float32),
                  jax.ShapeDtypeStruct((N_ACC, Wc), jnp.float32)),
        mesh=_mesh(),
        scratch_types=[
            pltpu.VMEM_SHARED((N_ACC, Wc), jnp.float32),
            pltpu.VMEM((B,), jnp.int32),
            pltpu.VMEM((B,), jnp.int32),
            pltpu.VMEM((B, Wc), jnp.float32),
            pltpu.SemaphoreType.DMA,
        ],
        compiler_params=pltpu.CompilerParams(use_tc_tiling_on_sc=False),
    )


WC = 96


def _spmm(*args):
    return _make_spmm(WC)(*args)


R = 400
GRID = (N // R,)
_HIGH = lax.Precision.HIGHEST


def _rows(w):
    return pl.BlockSpec((R, w), lambda i: (i, 0))


def _whole(shape):
    return pl.BlockSpec(shape, lambda i: (0,) * len(shape))


def _mm(a, b):
    return jnp.dot(a, b, preferred_element_type=jnp.float32, precision=_HIGH)


def _split(g, n, outs):
    for i, o in enumerate(outs):
        o[...] = g[:, i * n:(i + 1) * n]


def _tc1_body(x_ref, dg0, dg1, wcat, rcat, g0a, g0b, g0c, g0d, root, dinv_o):
    deg = dg0[:, :1] + dg1[:, :1]
    dinv = jnp.where(deg > 0, lax.rsqrt(jnp.maximum(deg, 1e-12)), 0.0)
    g = _mm(x_ref[...], wcat[...]) * dinv
    _split(g, WC, (g0a, g0b, g0c, g0d))
    root[...] = _mm(x_ref[...], rcat[...])
    dinv_o[...] = dinv


def _tc1(x, deg0, deg1, wcat, rcat):
    return pl.pallas_call(
        _tc1_body,
        grid=GRID,
        in_specs=[_rows(F_IN), _rows(DEG_W), _rows(DEG_W),
                  _whole((F_IN, 384)), _whole((F_IN, 384))],
        out_specs=(_rows(WC),) * 4 + (_rows(384), _rows(1)),
        out_shape=(jax.ShapeDtypeStruct((N, WC), jnp.float32),) * 4
                  + (jax.ShapeDtypeStruct((N, 384), jnp.float32),
                     jax.ShapeDtypeStruct((N, 1), jnp.float32)),
    )(x, deg0, deg1, wcat, rcat)


def _tc2_body(pa, pb, pc, pd, root, dinv, b_ref, wblk, ga, gb, gc, gd):
    p = jnp.concatenate([pa[...], pb[...], pc[...], pd[...]], axis=1)
    o = jnp.maximum(p * dinv[...] + root[...] + b_ref[...], 0.0)
    g = _mm(o, wblk[...]) * dinv[...]
    _split(g, WC, (ga, gb, gc, gd))


def _tc2(pa, pb, pc, pd, root, dinv, bcat, wblk):
    return pl.pallas_call(
        _tc2_body,
        grid=GRID,
        in_specs=[_rows(WC)] * 4 + [_rows(384), _rows(1),
                  _whole((1, 384)), _whole((384, 384))],
        out_specs=(_rows(WC),) * 4,
        out_shape=(jax.ShapeDtypeStruct((N, WC), jnp.float32),) * 4,
    )(pa, pb, pc, pd, root, dinv, bcat, wblk)


def _tc3_body(pa, pb, pc, pd, root, dinv, b_ref, w2cat, r2cat, g2a, g2b, root2):
    p = jnp.concatenate([pa[...], pb[...], pc[...], pd[...]], axis=1)
    o = jnp.maximum(p * dinv[...] + root[...] + b_ref[...], 0.0)
    h = (o[:, :128] + o[:, 128:256] + o[:, 256:384]) * (1.0 / 3.0)
    h = jnp.maximum(h, 0.0)
    g = _mm(h, w2cat[...]) * dinv[...]
    _split(g, WC, (g2a, g2b))
    root2[...] = _mm(h, r2cat[...])


def _tc3(pa, pb, pc, pd, root, dinv, bcat, w2cat, r2cat):
    return pl.pallas_call(
        _tc3_body,
        grid=GRID,
        in_specs=[_rows(WC)] * 4 + [_rows(384), _rows(1),
                  _whole((1, 384)), _whole((F_HID, 192)), _whole((F_HID, 192))],
        out_specs=(_rows(WC), _rows(WC), _rows(192)),
        out_shape=(jax.ShapeDtypeStruct((N, WC), jnp.float32),
                   jax.ShapeDtypeStruct((N, WC), jnp.float32),
                   jax.ShapeDtypeStruct((N, 192), jnp.float32)),
    )(pa, pb, pc, pd, root, dinv, bcat, w2cat, r2cat)


def _tc4_body(pa, pb, root2, dinv, b_ref, wblk, ga, gb):
    p = jnp.concatenate([pa[...], pb[...]], axis=1)
    o = p * dinv[...] + root2[...] + b_ref[...]
    g = _mm(o, wblk[...]) * dinv[...]
    ga[...] = g[:, :96]
    gb[...] = g[:, 96:]


def _tc4(pa, pb, root2, dinv, bcat, wblk):
    return pl.pallas_call(
        _tc4_body,
        grid=GRID,
        in_specs=[_rows(WC), _rows(WC), _rows(192), _rows(1),
                  _whole((1, 192)), _whole((192, 192))],
        out_specs=(_rows(WC), _rows(WC)),
        out_shape=(jax.ShapeDtypeStruct((N, 96), jnp.float32),
                   jax.ShapeDtypeStruct((N, 96), jnp.float32)),
    )(pa, pb, root2, dinv, bcat, wblk)


def _tc5_body(pa, pb, root2, dinv, b_ref, out):
    p = jnp.concatenate([pa[...], pb[...]], axis=1)
    o2 = p * dinv[...] + root2[...] + b_ref[...]
    o = (o2[:, :64] + o2[:, 64:128] + o2[:, 128:192]) * (1.0 / 3.0)
    m = jnp.max(o, axis=1, keepdims=True)
    ex = jnp.exp(o - m)
    lse = jnp.log(jnp.sum(ex, axis=1, keepdims=True)) + m
    out[...] = o - lse


def _tc5(pa, pb, root2, dinv, bcat):
    return pl.pallas_call(
        _tc5_body,
        grid=GRID,
        in_specs=[_rows(WC), _rows(WC), _rows(192), _rows(1), _whole((1, 192))],
        out_specs=_rows(F_OUT),
        out_shape=jax.ShapeDtypeStruct((N, F_OUT), jnp.float32),
    )(pa, pb, root2, dinv, bcat)


def _cat(w):
    return jnp.transpose(w, (1, 0, 2)).reshape(w.shape[1], K * w.shape[2])


def _blockdiag(w):
    fi, fo = w.shape[1], w.shape[2]
    z = jnp.zeros((K * fi, K * fo), w.dtype)
    for k in range(K):
        z = z.at[k * fi:(k + 1) * fi, k * fo:(k + 1) * fo].set(w[k])
    return z


def kernel(x, edge_index, init_w1, w1, root_w1, b1, init_w2, w2, root_w2, b2):
    w1cat = _cat(init_w1)
    r1cat = _cat(root_w1)
    w2cat = _cat(init_w2)
    r2cat = _cat(root_w2)
    w1blk = _blockdiag(w1)
    w2blk = _blockdiag(w2)
    b1cat = b1.reshape(1, K * F_HID)
    b2cat = b2.reshape(1, K * F_OUT)

    pad = EP - E
    row_p = jnp.concatenate([edge_index[0], jnp.zeros((pad,), jnp.int32)])
    col_p = jnp.concatenate([edge_index[1], jnp.full((pad,), N, jnp.int32)])

    deg0, deg1 = _deg(col_p)
    g0a, g0b, g0c, g0d, root1, dinv = _tc1(x, deg0, deg1, w1cat, r1cat)
    p0a, p0b = _spmm(g0a, g0b, row_p, col_p)
    p0c, p0d = _spmm(g0c, g0d, row_p, col_p)
    g1a, g1b, g1c, g1d = _tc2(p0a, p0b, p0c, p0d, root1, dinv, b1cat, w1blk)
    p1a, p1b = _spmm(g1a, g1b, row_p, col_p)
    p1c, p1d = _spmm(g1c, g1d, row_p, col_p)
    g2a, g2b, root2 = _tc3(p1a, p1b, p1c, p1d, root1, dinv, b1cat, w2cat, r2cat)
    p2a, p2b = _spmm(g2a, g2b, row_p, col_p)
    g3a, g3b = _tc4(p2a, p2b, root2, dinv, b2cat, w2blk)
    p3a, p3b = _spmm(g3a, g3b, row_p, col_p)
    return _tc5(p3a, p3b, root2, dinv, b2cat)
